# baseline (speedup 1.0000x reference)
import functools

import jax
import jax.numpy as jnp
from jax import lax
from jax.experimental import pallas as pl
from jax.experimental.pallas import tpu as pltpu

_K = 32
_BLOCK_Q = 256


def _body(ph_ref, pl_ref, sh_ref, sl_ref, x_ref, ones_ref, out_ref):
    ph = ph_ref[...]
    plq = pl_ref[...]
    x = x_ref[...]

    dot = jax.lax.dot_general(
        ph, plq, (((0,), (0,)), ((), ())),
        preferred_element_type=jnp.float32)
    d2 = sh_ref[...].T + sl_ref[...] - 2.0 * dot

    s = jax.lax.bitcast_convert_type(d2 + 4.0, jnp.int32)

    r = s.shape[0]
    lo0 = jnp.full((r, 1), 0x40600000, dtype=jnp.int32)
    hi0 = jnp.full((r, 1), 0x40E00000, dtype=jnp.int32)

    def step(_, carry):
        lo, hi = carry
        mid = lo + ((hi - lo) >> 1)
        t = jnp.where(s <= mid, 1.0, 0.0)
        cnt = lax.dot_general(
            t, ones_ref[...], (((1,), (0,)), ((), ())),
            preferred_element_type=jnp.float32)[:, 0:1]
        ge = cnt >= float(_K)
        return jnp.where(ge, lo, mid + 1), jnp.where(ge, mid, hi)

    _, thresh = jax.lax.fori_loop(0, 23, step, (lo0, hi0))

    mask = s <= thresh
    w = jnp.where(mask, 1.0 / jnp.maximum(d2, 1e-16), 0.0)
    den = jnp.sum(w, axis=1, keepdims=True)
    num = jax.lax.dot_general(
        w, x, (((1,), (0,)), ((), ())),
        precision=jax.lax.Precision.HIGHEST,
        preferred_element_type=jnp.float32)
    out_ref[...] = num / den


def kernel(x, pos_l, pos_h):
    nl, d = x.shape
    nq = pos_h.shape[0]

    sq_l = jnp.sum(pos_l * pos_l, axis=-1)[None, :]
    sq_h = jnp.sum(pos_h * pos_h, axis=-1, keepdims=True).T
    plt = jnp.concatenate(
        [pos_l.T, jnp.zeros((5, nl), jnp.float32)], axis=0)
    pht = jnp.concatenate(
        [pos_h.T, jnp.zeros((5, nq), jnp.float32)], axis=0)

    bq = min(_BLOCK_Q, nq)
    grid = (nq // bq,)
    return pl.pallas_call(
        _body,
        grid=grid,
        in_specs=[
            pl.BlockSpec((8, bq), lambda i: (0, i)),
            pl.BlockSpec((8, nl), lambda i: (0, 0)),
            pl.BlockSpec((1, bq), lambda i: (0, i)),
            pl.BlockSpec((1, nl), lambda i: (0, 0)),
            pl.BlockSpec((nl, d), lambda i: (0, 0)),
            pl.BlockSpec((nl, 128), lambda i: (0, 0)),
        ],
        out_specs=pl.BlockSpec((bq, d), lambda i: (i, 0)),
        out_shape=jax.ShapeDtypeStruct((nq, d), jnp.float32),
        compiler_params=pltpu.CompilerParams(
            dimension_semantics=("arbitrary",)),
    )(pht, plt, sq_h, sq_l, x, jnp.ones((nl, 128), jnp.float32))


# speedup vs baseline: 1.3795x; 1.3795x over previous
import functools

import jax
import jax.numpy as jnp
from jax import lax
from jax.experimental import pallas as pl
from jax.experimental.pallas import tpu as pltpu

_K = 32
_BLOCK_Q = 512


def _body(ph_ref, pl_ref, sh_ref, sl_ref, x_ref, out_ref):
    ph = ph_ref[...]
    plq = pl_ref[...]
    x = x_ref[...]

    dot = jax.lax.dot_general(
        ph, plq, (((0,), (0,)), ((), ())),
        preferred_element_type=jnp.float32)
    d2 = sh_ref[...].T + sl_ref[...] - 2.0 * dot

    s = jax.lax.bitcast_convert_type(d2 + 4.0, jnp.int32)

    r = s.shape[0]
    lo0 = jnp.full((r, 1), 0x40600000, dtype=jnp.int32)
    hi0 = jnp.full((r, 1), 0x40E00000, dtype=jnp.int32)

    def step(_, carry):
        lo, hi = carry
        mid = lo + ((hi - lo) >> 1)
        t = jnp.where(s <= mid, 1.0, 0.0)
        while t.shape[1] > 128:
            h = t.shape[1] // 2
            t = t[:, :h] + t[:, h:]
        cnt = jnp.sum(t, axis=1, keepdims=True)
        ge = cnt >= float(_K)
        return jnp.where(ge, lo, mid + 1), jnp.where(ge, mid, hi)

    _, thresh = jax.lax.fori_loop(0, 23, step, (lo0, hi0))

    mask = s <= thresh
    w = jnp.where(mask, 1.0 / jnp.maximum(d2, 1e-16), 0.0)
    den = jnp.sum(w, axis=1, keepdims=True)
    num = jax.lax.dot_general(
        w, x, (((1,), (0,)), ((), ())),
        precision=jax.lax.Precision.HIGHEST,
        preferred_element_type=jnp.float32)
    out_ref[...] = num / den


def kernel(x, pos_l, pos_h):
    nl, d = x.shape
    nq = pos_h.shape[0]

    sq_l = jnp.sum(pos_l * pos_l, axis=-1)[None, :]
    sq_h = jnp.sum(pos_h * pos_h, axis=-1, keepdims=True).T
    plt = jnp.concatenate(
        [pos_l.T, jnp.zeros((5, nl), jnp.float32)], axis=0)
    pht = jnp.concatenate(
        [pos_h.T, jnp.zeros((5, nq), jnp.float32)], axis=0)

    bq = min(_BLOCK_Q, nq)
    grid = (nq // bq,)
    return pl.pallas_call(
        _body,
        grid=grid,
        in_specs=[
            pl.BlockSpec((8, bq), lambda i: (0, i)),
            pl.BlockSpec((8, nl), lambda i: (0, 0)),
            pl.BlockSpec((1, bq), lambda i: (0, i)),
            pl.BlockSpec((1, nl), lambda i: (0, 0)),
            pl.BlockSpec((nl, d), lambda i: (0, 0)),
        ],
        out_specs=pl.BlockSpec((bq, d), lambda i: (i, 0)),
        out_shape=jax.ShapeDtypeStruct((nq, d), jnp.float32),
        compiler_params=pltpu.CompilerParams(
            dimension_semantics=("arbitrary",)),
    )(pht, plt, sq_h, sq_l, x)


# speedup vs baseline: 1.4025x; 1.0167x over previous
import functools

import jax
import jax.numpy as jnp
from jax import lax
from jax.experimental import pallas as pl
from jax.experimental.pallas import tpu as pltpu

_K = 32
_BLOCK_Q = 1024


def _body(ph_ref, pl_ref, sh_ref, sl_ref, x_ref, out_ref):
    ph = ph_ref[...]
    plq = pl_ref[...]
    x = x_ref[...]

    dot = jax.lax.dot_general(
        ph, plq, (((0,), (0,)), ((), ())),
        preferred_element_type=jnp.float32)
    d2 = sh_ref[...].T + sl_ref[...] - 2.0 * dot

    s = jax.lax.bitcast_convert_type(d2 + 4.0, jnp.int32)

    r = s.shape[0]
    lo0 = jnp.full((r, 1), 0x40600000, dtype=jnp.int32)
    hi0 = jnp.full((r, 1), 0x40E00000, dtype=jnp.int32)

    def step(_, carry):
        lo, hi = carry
        mid = lo + ((hi - lo) >> 1)
        t = jnp.where(s <= mid, 1.0, 0.0)
        while t.shape[1] > 128:
            h = t.shape[1] // 2
            t = t[:, :h] + t[:, h:]
        cnt = jnp.sum(t, axis=1, keepdims=True)
        ge = cnt >= float(_K)
        return jnp.where(ge, lo, mid + 1), jnp.where(ge, mid, hi)

    _, thresh = jax.lax.fori_loop(0, 23, step, (lo0, hi0))

    mask = s <= thresh
    w = jnp.where(mask, 1.0 / jnp.maximum(d2, 1e-16), 0.0)
    den = jnp.sum(w, axis=1, keepdims=True)
    num = jax.lax.dot_general(
        w, x, (((1,), (0,)), ((), ())),
        precision=jax.lax.Precision.HIGHEST,
        preferred_element_type=jnp.float32)
    out_ref[...] = num / den


def kernel(x, pos_l, pos_h):
    nl, d = x.shape
    nq = pos_h.shape[0]

    sq_l = jnp.sum(pos_l * pos_l, axis=-1)[None, :]
    sq_h = jnp.sum(pos_h * pos_h, axis=-1, keepdims=True).T
    plt = jnp.concatenate(
        [pos_l.T, jnp.zeros((5, nl), jnp.float32)], axis=0)
    pht = jnp.concatenate(
        [pos_h.T, jnp.zeros((5, nq), jnp.float32)], axis=0)

    bq = min(_BLOCK_Q, nq)
    grid = (nq // bq,)
    return pl.pallas_call(
        _body,
        grid=grid,
        in_specs=[
            pl.BlockSpec((8, bq), lambda i: (0, i)),
            pl.BlockSpec((8, nl), lambda i: (0, 0)),
            pl.BlockSpec((1, bq), lambda i: (0, i)),
            pl.BlockSpec((1, nl), lambda i: (0, 0)),
            pl.BlockSpec((nl, d), lambda i: (0, 0)),
        ],
        out_specs=pl.BlockSpec((bq, d), lambda i: (i, 0)),
        out_shape=jax.ShapeDtypeStruct((nq, d), jnp.float32),
        compiler_params=pltpu.CompilerParams(
            dimension_semantics=("arbitrary",)),
    )(pht, plt, sq_h, sq_l, x)


# speedup vs baseline: 1.6533x; 1.1788x over previous
import functools

import jax
import jax.numpy as jnp
from jax import lax
from jax.experimental import pallas as pl
from jax.experimental.pallas import tpu as pltpu

_K = 32
_BLOCK_Q = 1024


def _body(ph_ref, pl_ref, sh_ref, sl_ref, x_ref, out_ref):
    ph = ph_ref[...]
    plq = pl_ref[...]
    x = x_ref[...]

    dot = jax.lax.dot_general(
        ph, plq, (((0,), (0,)), ((), ())),
        preferred_element_type=jnp.float32)
    d2 = sh_ref[...].T + sl_ref[...] - 2.0 * dot

    s = jax.lax.bitcast_convert_type(d2 + 4.0, jnp.int32)

    r = s.shape[0]
    lo0 = jnp.full((r, 1), 0x40600000, dtype=jnp.int32)
    hi0 = jnp.full((r, 1), 0x40E00000, dtype=jnp.int32)

    def step(_, carry):
        lo, hi = carry
        mid = lo + ((hi - lo) >> 1)
        t = jnp.where(s <= mid, 1.0, 0.0)
        while t.shape[1] > 128:
            h = t.shape[1] // 2
            t = t[:, :h] + t[:, h:]
        cnt = jnp.sum(t, axis=1, keepdims=True)
        ge = cnt >= float(_K)
        return jnp.where(ge, lo, mid + 1), jnp.where(ge, mid, hi)

    _, thresh = jax.lax.fori_loop(0, 23, step, (lo0, hi0))

    mask = s <= thresh
    w = jnp.where(mask, 1.0 / jnp.maximum(d2, 1e-16), 0.0)
    den = jnp.sum(w, axis=1, keepdims=True)
    num = jax.lax.dot_general(
        w, x, (((1,), (0,)), ((), ())),
        preferred_element_type=jnp.float32)
    out_ref[...] = num / den


def kernel(x, pos_l, pos_h):
    nl, d = x.shape
    nq = pos_h.shape[0]

    sq_l = jnp.sum(pos_l * pos_l, axis=-1)[None, :]
    sq_h = jnp.sum(pos_h * pos_h, axis=-1, keepdims=True).T
    plt = jnp.concatenate(
        [pos_l.T, jnp.zeros((5, nl), jnp.float32)], axis=0)
    pht = jnp.concatenate(
        [pos_h.T, jnp.zeros((5, nq), jnp.float32)], axis=0)

    bq = min(_BLOCK_Q, nq)
    grid = (nq // bq,)
    return pl.pallas_call(
        _body,
        grid=grid,
        in_specs=[
            pl.BlockSpec((8, bq), lambda i: (0, i)),
            pl.BlockSpec((8, nl), lambda i: (0, 0)),
            pl.BlockSpec((1, bq), lambda i: (0, i)),
            pl.BlockSpec((1, nl), lambda i: (0, 0)),
            pl.BlockSpec((nl, d), lambda i: (0, 0)),
        ],
        out_specs=pl.BlockSpec((bq, d), lambda i: (i, 0)),
        out_shape=jax.ShapeDtypeStruct((nq, d), jnp.float32),
        compiler_params=pltpu.CompilerParams(
            dimension_semantics=("arbitrary",)),
    )(pht, plt, sq_h, sq_l, x)
